# Initial kernel scaffold; baseline (speedup 1.0000x reference)
import jax
import jax.numpy as jnp
from jax.experimental import pallas as pl


def kernel(h, node_feat, W, b):
    raise NotImplementedError("write your pallas kernel here")


# speedup vs baseline: 1.2449x; 1.2449x over previous
import functools

import jax
import jax.numpy as jnp
from jax.experimental import pallas as pl

N = 10000
DEG = 32
D = 128
OUT = 128
BN = 400


def _pna_kernel(h_ref, nf_ref, w_ref, b_ref, out_ref):
    h = h_ref[...]
    inv = 1.0 / DEG
    s = jnp.sum(h, axis=1)
    mean = s * inv
    mn = jnp.min(h, axis=1)
    mx = jnp.max(h, axis=1)
    var = jnp.sum(h * h, axis=1) * inv - mean * mean
    std = jnp.sqrt(jax.nn.relu(var) + 1e-5)
    w = w_ref[...]
    acc = jnp.dot(mean, w[0:D], preferred_element_type=jnp.float32)
    acc += jnp.dot(mn, w[D:2 * D], preferred_element_type=jnp.float32)
    acc += jnp.dot(mx, w[2 * D:3 * D], preferred_element_type=jnp.float32)
    acc += jnp.dot(std, w[3 * D:4 * D], preferred_element_type=jnp.float32)
    acc += jnp.dot(nf_ref[...], w[4 * D:5 * D],
                   preferred_element_type=jnp.float32)
    out_ref[...] = acc + b_ref[...]


@jax.jit
def kernel(h, node_feat, W, b):
    b2 = b.reshape(1, OUT)
    grid = (N // BN,)
    return pl.pallas_call(
        _pna_kernel,
        grid=grid,
        in_specs=[
            pl.BlockSpec((BN, DEG, D), lambda i: (i, 0, 0)),
            pl.BlockSpec((BN, D), lambda i: (i, 0)),
            pl.BlockSpec((5 * D, OUT), lambda i: (0, 0)),
            pl.BlockSpec((1, OUT), lambda i: (0, 0)),
        ],
        out_specs=pl.BlockSpec((BN, OUT), lambda i: (i, 0)),
        out_shape=jax.ShapeDtypeStruct((N, OUT), jnp.float32),
    )(h, node_feat, W, b2)
